# Initial kernel scaffold; baseline (speedup 1.0000x reference)
import jax
import jax.numpy as jnp
from jax.experimental import pallas as pl


def kernel(x, Wq, Wk, Wv, Wo):
    raise NotImplementedError("write your pallas kernel here")


# speedup vs baseline: 10.8685x; 10.8685x over previous
import functools
import math

import numpy as np
import jax
import jax.numpy as jnp
from jax import lax
from jax.experimental import pallas as pl
from jax.experimental.pallas import tpu as pltpu
from jax.experimental.pallas import tpu_sc as plsc

B, S, DM = 1, 4096, 768
QH, KVH = 12, 4
DH = DM // QH
HALF = DH // 2
M = 64
TOPK = 4

SB = 512
NBLK = S // SB

_inv_freq = 1.0 / (10000.0 ** (np.arange(0, HALF, dtype=np.float32) / HALF))
_freqs = np.outer(np.arange(S, dtype=np.float32), _inv_freq)
_COS_NP = np.cos(_freqs).astype(np.float32)
_SIN_NP = np.sin(_freqs).astype(np.float32)


def _kv_body(x_ref, wk_ref, wv_ref, cos_ref, sin_ref, v_ref, k1_ref, k2_ref):
    i = pl.program_id(0)
    xb = x_ref[...]
    cos = cos_ref[...]
    sin = sin_ref[...]

    for h in range(KVH):
        kh = jnp.dot(xb, wk_ref[:, h * DH:(h + 1) * DH],
                     preferred_element_type=jnp.float32)
        kh1 = kh[:, :HALF] * cos - kh[:, HALF:] * sin
        kh2 = kh[:, HALF:] * cos + kh[:, :HALF] * sin
        c1 = jnp.concatenate(
            [jnp.sum(kh1[m * M:(m + 1) * M], axis=0, keepdims=True)
             for m in range(SB // M)], axis=0)
        c2 = kh2[0 * M:1 * M]
        for m in range(1, SB // M):
            c2 = c2 + kh2[m * M:(m + 1) * M]
        k1_ref[h] = c1

        @pl.when(i == 0)
        def _(h=h, c2=c2):
            k2_ref[h] = c2

        @pl.when(i != 0)
        def _(h=h, c2=c2):
            k2_ref[h] += c2

        v_ref[h] = jnp.dot(xb, wv_ref[:, h * DH:(h + 1) * DH],
                           preferred_element_type=jnp.float32)


def _stage_a(x2, Wk, Wv, cos, sin):
    return pl.pallas_call(
        _kv_body,
        grid=(NBLK,),
        in_specs=[
            pl.BlockSpec((SB, DM), lambda i: (i, 0)),
            pl.BlockSpec((DM, KVH * DH), lambda i: (0, 0)),
            pl.BlockSpec((DM, KVH * DH), lambda i: (0, 0)),
            pl.BlockSpec((SB, HALF), lambda i: (i, 0)),
            pl.BlockSpec((SB, HALF), lambda i: (i, 0)),
        ],
        out_specs=[
            pl.BlockSpec((KVH, SB, DH), lambda i: (0, i, 0)),
            pl.BlockSpec((KVH, SB // M, HALF), lambda i: (0, i, 0)),
            pl.BlockSpec((KVH, M, HALF), lambda i: (0, 0, 0)),
        ],
        out_shape=[
            jax.ShapeDtypeStruct((KVH, S, DH), jnp.float32),
            jax.ShapeDtypeStruct((KVH, M, HALF), jnp.float32),
            jax.ShapeDtypeStruct((KVH, M, HALF), jnp.float32),
        ],
    )(x2, Wk, Wv, cos, sin)


def _top4(s, n):
    io = lax.broadcasted_iota(jnp.int32, s.shape, 1)
    vals, idxs = [], []
    for _ in range(TOPK):
        m = jnp.max(s, axis=1, keepdims=True)
        idx = jnp.min(jnp.where(s == m, io, n), axis=1, keepdims=True)
        vals.append(m)
        idxs.append(idx)
        s = jnp.where(io == idx, -jnp.inf, s)
    return jnp.concatenate(vals, axis=1), jnp.concatenate(idxs, axis=1)


def _sel4(tab, sel):
    out = jnp.zeros_like(tab)
    for t in range(TOPK):
        out = jnp.where(sel == t, tab[:, t:t + 1], out)
    return out


def _q_body(x_ref, wq_ref, k1_ref, k2_ref, cos_ref, sin_ref, w_ref, fidx_ref):
    xb = x_ref[...]
    xq = jnp.dot(xb, wq_ref[...], preferred_element_type=jnp.float32)
    cos = cos_ref[...]
    sin = sin_ref[...]
    dn = (((1,), (1,)), ((), ()))
    for h in range(QH):
        qh = xq[:, h * DH:(h + 1) * DH]
        q1 = qh[:, :HALF] * cos - qh[:, HALF:] * sin
        q2 = qh[:, HALF:] * cos + qh[:, :HALF] * sin
        kv = h // (QH // KVH)
        k1h = k1_ref[kv]
        k2h = k2_ref[kv]
        s1 = lax.dot_general(q1, k1h, dn, preferred_element_type=jnp.float32)
        s2 = lax.dot_general(q2, k2h, dn, preferred_element_type=jnp.float32)
        s1e = lax.dot_general(q1, k1h, dn, precision=lax.Precision.HIGHEST,
                              preferred_element_type=jnp.float32)
        s2e = lax.dot_general(q2, k2h, dn, precision=lax.Precision.HIGHEST,
                              preferred_element_type=jnp.float32)
        _, i1 = _top4(s1, M)
        _, i2 = _top4(s2, M)
        io = lax.broadcasted_iota(jnp.int32, (SB, M), 1)
        v1 = jnp.concatenate(
            [jnp.sum(jnp.where(io == i1[:, t:t + 1], s1e, 0.0),
                     axis=1, keepdims=True) for t in range(TOPK)], axis=1)
        v2 = jnp.concatenate(
            [jnp.sum(jnp.where(io == i2[:, t:t + 1], s2e, 0.0),
                     axis=1, keepdims=True) for t in range(TOPK)], axis=1)
        comb = jnp.concatenate(
            [v1[:, t:t + 1] + v2 for t in range(TOPK)], axis=1)
        cv, sel = _top4(comb, TOPK * TOPK)
        a = cv * (1.0 / math.sqrt(DH))
        e = jnp.exp(a - jnp.max(a, axis=1, keepdims=True))
        w = e / jnp.sum(e, axis=1, keepdims=True)
        idx1 = sel // TOPK
        idx2 = sel % TOPK
        row = _sel4(i1, idx1)
        col = _sel4(i2, idx2)
        w_ref[h] = w
        fidx_ref[h] = row * M + col + kv * S


def _stage_b(x2, Wq, k1, k2, cos, sin):
    return pl.pallas_call(
        _q_body,
        grid=(NBLK,),
        in_specs=[
            pl.BlockSpec((SB, DM), lambda i: (i, 0)),
            pl.BlockSpec((DM, QH * DH), lambda i: (0, 0)),
            pl.BlockSpec((KVH, M, HALF), lambda i: (0, 0, 0)),
            pl.BlockSpec((KVH, M, HALF), lambda i: (0, 0, 0)),
            pl.BlockSpec((SB, HALF), lambda i: (i, 0)),
            pl.BlockSpec((SB, HALF), lambda i: (i, 0)),
        ],
        out_specs=[
            pl.BlockSpec((QH, SB, TOPK), lambda i: (0, i, 0)),
            pl.BlockSpec((QH, SB, TOPK), lambda i: (0, i, 0)),
        ],
        out_shape=[
            jax.ShapeDtypeStruct((QH, S, TOPK), jnp.float32),
            jax.ShapeDtypeStruct((QH, S, TOPK), jnp.int32),
        ],
    )(x2, Wq, k1, k2, cos, sin)


NQ_TOT = QH * S
NW = 32
NQ_W = NQ_TOT // NW
CH = 128
NCHUNK = NQ_W // CH
GSUB = (CH * TOPK) // 128


def _sc_body(fidx_hbm, w_hbm, vtab_hbm, out_hbm,
             idx_raw, idx2_v, off_v, w_v, rows_v, out_v, sem):
    wid = lax.axis_index("s") * 2 + lax.axis_index("c")
    qbase = wid * NQ_W

    def chunk(ci, carry):
        q0 = qbase + ci * CH
        for g in range(GSUB):
            pltpu.sync_copy(fidx_hbm.at[pl.ds(q0 * TOPK + g * 128, 128)],
                            idx_raw.at[g])
        pltpu.sync_copy(w_hbm.at[pl.ds(q0 * TOPK, CH * TOPK)],
                        w_v.at[pl.ds(0, CH * TOPK)])
        for g in range(GSUB):
            for j in range(128 // 16):
                sl = pl.ds(j * 16, 16)
                raw = idx_raw[g, sl]
                idx2_v[g, sl] = raw >> 1
                off_v[pl.ds(g * 128 + j * 16, 16)] = (raw & 1) * DH
        copies = [
            pltpu.async_copy(vtab_hbm.at[idx2_v.at[g]],
                             rows_v.at[pl.ds(g * 128, 128)], sem)
            for g in range(GSUB)
        ]
        for c in copies:
            c.wait()

        def qloop(qi, c2):
            wv4 = w_v[pl.ds(qi * TOPK, 16)]
            of4 = off_v[pl.ds(qi * TOPK, 16)]
            for dv in range(DH // 16):
                acc = wv4[0] * rows_v[qi * TOPK, pl.ds(of4[0] + dv * 16, 16)]
                for kk in range(1, TOPK):
                    acc = acc + wv4[kk] * rows_v[
                        qi * TOPK + kk, pl.ds(of4[kk] + dv * 16, 16)]
                out_v[qi, pl.ds(dv * 16, 16)] = acc
            return c2

        lax.fori_loop(0, CH, qloop, 0)
        pltpu.sync_copy(out_v, out_hbm.at[pl.ds(q0, CH)])
        return carry

    lax.fori_loop(0, NCHUNK, chunk, 0)


@functools.lru_cache(maxsize=1)
def _make_sc_gather():
    return functools.partial(
        pl.kernel,
        out_type=jax.ShapeDtypeStruct((NQ_TOT, DH), jnp.float32),
        mesh=plsc.VectorSubcoreMesh(core_axis_name="c", subcore_axis_name="s"),
        scratch_types=[
            pltpu.VMEM((GSUB, 128), jnp.int32),
            pltpu.VMEM((GSUB, 128), jnp.int32),
            pltpu.VMEM((CH * TOPK + 16,), jnp.int32),
            pltpu.VMEM((CH * TOPK + 16,), jnp.float32),
            pltpu.VMEM((CH * TOPK, 2 * DH), jnp.float32),
            pltpu.VMEM((CH, DH), jnp.float32),
            pltpu.SemaphoreType.DMA,
        ],
    )(_sc_body)


def _o_body(ctx_ref, wo_ref, out_ref):
    acc = jnp.dot(ctx_ref[0], wo_ref[0], preferred_element_type=jnp.float32)
    for h in range(1, QH):
        acc += jnp.dot(ctx_ref[h], wo_ref[h],
                       preferred_element_type=jnp.float32)
    out_ref[...] = acc


def _stage_d(ctx3, Wo):
    return pl.pallas_call(
        _o_body,
        grid=(NBLK,),
        in_specs=[
            pl.BlockSpec((QH, SB, DH), lambda i: (0, i, 0)),
            pl.BlockSpec((QH, DH, DM), lambda i: (0, 0, 0)),
        ],
        out_specs=pl.BlockSpec((SB, DM), lambda i: (i, 0)),
        out_shape=jax.ShapeDtypeStruct((S, DM), jnp.float32),
    )(ctx3, Wo.reshape(QH, DH, DM))


def kernel(x, Wq, Wk, Wv, Wo):
    x2 = x.reshape(S, DM)
    cos = jnp.asarray(_COS_NP)
    sin = jnp.asarray(_SIN_NP)
    v_tab, k1, k2 = _stage_a(x2, Wk, Wv, cos, sin)
    w, fidx = _stage_b(x2, Wq, k1, k2, cos, sin)
    fidx_flat = fidx.reshape(NQ_TOT * TOPK)
    w_flat = w.reshape(NQ_TOT * TOPK)
    v_pack = v_tab.reshape(KVH * S // 2, 2 * DH)
    ctx = _make_sc_gather()(fidx_flat, w_flat, v_pack)
    ctx3 = ctx.reshape(QH, S, DH)
    out = _stage_d(ctx3, Wo)
    return out.reshape(B, S, DM)


# speedup vs baseline: 32.2348x; 2.9659x over previous
import functools
import math

import numpy as np
import jax
import jax.numpy as jnp
from jax import lax
from jax.experimental import pallas as pl
from jax.experimental.pallas import tpu as pltpu
from jax.experimental.pallas import tpu_sc as plsc

B, S, DM = 1, 4096, 768
QH, KVH = 12, 4
DH = DM // QH
HALF = DH // 2
M = 64
TOPK = 4

SB = 512
NBLK = S // SB

_inv_freq = 1.0 / (10000.0 ** (np.arange(0, HALF, dtype=np.float32) / HALF))
_freqs = np.outer(np.arange(S, dtype=np.float32), _inv_freq)
_COS_NP = np.cos(_freqs).astype(np.float32)
_SIN_NP = np.sin(_freqs).astype(np.float32)


def _kv_body(x_ref, wk_ref, wv_ref, cos_ref, sin_ref, v_ref, k1_ref, k2_ref):
    i = pl.program_id(0)
    xb = x_ref[...]
    cos = cos_ref[...]
    sin = sin_ref[...]

    for h in range(KVH):
        kh = jnp.dot(xb, wk_ref[:, h * DH:(h + 1) * DH],
                     preferred_element_type=jnp.float32)
        kh1 = kh[:, :HALF] * cos - kh[:, HALF:] * sin
        kh2 = kh[:, HALF:] * cos + kh[:, :HALF] * sin
        c1 = jnp.concatenate(
            [jnp.sum(kh1[m * M:(m + 1) * M], axis=0, keepdims=True)
             for m in range(SB // M)], axis=0)
        c2 = kh2[0 * M:1 * M]
        for m in range(1, SB // M):
            c2 = c2 + kh2[m * M:(m + 1) * M]
        k1_ref[h] = c1

        @pl.when(i == 0)
        def _(h=h, c2=c2):
            k2_ref[h] = c2

        @pl.when(i != 0)
        def _(h=h, c2=c2):
            k2_ref[h] += c2

        v_ref[h] = jnp.dot(xb, wv_ref[:, h * DH:(h + 1) * DH],
                           preferred_element_type=jnp.float32)


def _stage_a(x2, Wk, Wv, cos, sin):
    return pl.pallas_call(
        _kv_body,
        grid=(NBLK,),
        in_specs=[
            pl.BlockSpec((SB, DM), lambda i: (i, 0)),
            pl.BlockSpec((DM, KVH * DH), lambda i: (0, 0)),
            pl.BlockSpec((DM, KVH * DH), lambda i: (0, 0)),
            pl.BlockSpec((SB, HALF), lambda i: (i, 0)),
            pl.BlockSpec((SB, HALF), lambda i: (i, 0)),
        ],
        out_specs=[
            pl.BlockSpec((KVH, SB, DH), lambda i: (0, i, 0)),
            pl.BlockSpec((KVH, SB // M, HALF), lambda i: (0, i, 0)),
            pl.BlockSpec((KVH, M, HALF), lambda i: (0, 0, 0)),
        ],
        out_shape=[
            jax.ShapeDtypeStruct((KVH, S, DH), jnp.float32),
            jax.ShapeDtypeStruct((KVH, M, HALF), jnp.float32),
            jax.ShapeDtypeStruct((KVH, M, HALF), jnp.float32),
        ],
    )(x2, Wk, Wv, cos, sin)


def _top4_t(s, se, n):
    io = lax.broadcasted_iota(jnp.int32, s.shape, 0)
    vals, idxs = [], []
    for _ in range(TOPK):
        m = jnp.max(s, axis=0, keepdims=True)
        idx = jnp.min(jnp.where(s == m, io, n), axis=0, keepdims=True)
        hit = io == idx
        if se is None:
            vals.append(m)
        else:
            vals.append(jnp.sum(jnp.where(hit, se, 0.0), axis=0,
                                keepdims=True))
        idxs.append(idx)
        s = jnp.where(hit, -jnp.inf, s)
    return jnp.concatenate(vals, axis=0), jnp.concatenate(idxs, axis=0)


def _sel4_t(tab, sel):
    out = jnp.zeros_like(tab)
    for t in range(TOPK):
        out = jnp.where(sel == t, tab[t:t + 1], out)
    return out


def _q_body(x_ref, wq_ref, k1_ref, k2_ref, cos_ref, sin_ref, w_ref, fidx_ref):
    xb = x_ref[...]
    xq = jnp.dot(xb, wq_ref[...], preferred_element_type=jnp.float32)
    cos = cos_ref[...]
    sin = sin_ref[...]
    dn = (((1,), (1,)), ((), ()))
    for h in range(QH):
        qh = xq[:, h * DH:(h + 1) * DH]
        q1 = qh[:, :HALF] * cos - qh[:, HALF:] * sin
        q2 = qh[:, HALF:] * cos + qh[:, :HALF] * sin
        kv = h // (QH // KVH)
        k1h = k1_ref[kv]
        k2h = k2_ref[kv]
        s1 = lax.dot_general(k1h, q1, dn, preferred_element_type=jnp.float32)
        s2 = lax.dot_general(k2h, q2, dn, preferred_element_type=jnp.float32)
        s1e = lax.dot_general(k1h, q1, dn, precision=lax.Precision.HIGHEST,
                              preferred_element_type=jnp.float32)
        s2e = lax.dot_general(k2h, q2, dn, precision=lax.Precision.HIGHEST,
                              preferred_element_type=jnp.float32)
        v1, i1 = _top4_t(s1, s1e, M)
        v2, i2 = _top4_t(s2, s2e, M)
        comb = jnp.concatenate(
            [v1[t:t + 1] + v2 for t in range(TOPK)], axis=0)
        cv, sel = _top4_t(comb, None, TOPK * TOPK)
        a = cv * (1.0 / math.sqrt(DH))
        e = jnp.exp(a - jnp.max(a, axis=0, keepdims=True))
        w = e / jnp.sum(e, axis=0, keepdims=True)
        idx1 = sel // TOPK
        idx2 = sel % TOPK
        row = _sel4_t(i1, idx1)
        col = _sel4_t(i2, idx2)
        w_ref[h] = w
        fidx_ref[h] = row * M + col + kv * S


def _stage_b(x2, Wq, k1, k2, cos, sin):
    return pl.pallas_call(
        _q_body,
        grid=(NBLK,),
        in_specs=[
            pl.BlockSpec((SB, DM), lambda i: (i, 0)),
            pl.BlockSpec((DM, QH * DH), lambda i: (0, 0)),
            pl.BlockSpec((KVH, M, HALF), lambda i: (0, 0, 0)),
            pl.BlockSpec((KVH, M, HALF), lambda i: (0, 0, 0)),
            pl.BlockSpec((SB, HALF), lambda i: (i, 0)),
            pl.BlockSpec((SB, HALF), lambda i: (i, 0)),
        ],
        out_specs=[
            pl.BlockSpec((QH, TOPK, SB), lambda i: (0, 0, i)),
            pl.BlockSpec((QH, TOPK, SB), lambda i: (0, 0, i)),
        ],
        out_shape=[
            jax.ShapeDtypeStruct((QH, TOPK, S), jnp.float32),
            jax.ShapeDtypeStruct((QH, TOPK, S), jnp.int32),
        ],
    )(x2, Wq, k1, k2, cos, sin)


NQ_TOT = QH * S
NW = 32
NQ_W = NQ_TOT // NW
CH = 128
NCHUNK = NQ_W // CH
GSUB = (CH * TOPK) // 128


def _sc_body(fidx_hbm, w_hbm, vtab_hbm, out_hbm,
             idx_raw, idx2_v, w2d_v, off_v, w_v, rows_v, out_v, sem):
    wid = lax.axis_index("s") * 2 + lax.axis_index("c")
    qbase = wid * NQ_W

    def chunk(ci, carry):
        q0 = qbase + ci * CH
        h = q0 // S
        s0 = pl.multiple_of(q0 % S, CH)
        pltpu.sync_copy(fidx_hbm.at[h, :, pl.ds(s0, CH)], idx_raw)
        pltpu.sync_copy(w_hbm.at[h, :, pl.ds(s0, CH)], w2d_v)
        for k in range(TOPK):
            for j in range(CH // 16):
                sl = pl.ds(j * 16, 16)
                fl = pl.ds(k * CH + j * 16, 16)
                raw = idx_raw[k, sl]
                idx2_v[k, sl] = raw >> 1
                off_v[fl] = (raw & 1) * DH
                w_v[fl] = w2d_v[k, sl]
        copies = [
            pltpu.async_copy(vtab_hbm.at[idx2_v.at[k]],
                             rows_v.at[pl.ds(k * CH, CH)], sem)
            for k in range(TOPK)
        ]
        for c in copies:
            c.wait()

        def qloop(qi, c2):
            acc = [None] * (DH // 16)
            for kk in range(TOPK):
                wk = w_v[pl.ds(kk * CH + qi, 16)][0]
                ofk = off_v[pl.ds(kk * CH + qi, 16)][0]
                r = kk * CH + qi
                for dv in range(DH // 16):
                    term = wk * rows_v[r, pl.ds(ofk + dv * 16, 16)]
                    acc[dv] = term if kk == 0 else acc[dv] + term
            for dv in range(DH // 16):
                out_v[qi, pl.ds(dv * 16, 16)] = acc[dv]
            return c2

        lax.fori_loop(0, CH, qloop, 0)
        pltpu.sync_copy(out_v, out_hbm.at[pl.ds(q0, CH)])
        return carry

    lax.fori_loop(0, NCHUNK, chunk, 0)


@functools.lru_cache(maxsize=1)
def _make_sc_gather():
    return functools.partial(
        pl.kernel,
        out_type=jax.ShapeDtypeStruct((NQ_TOT, DH), jnp.float32),
        mesh=plsc.VectorSubcoreMesh(core_axis_name="c", subcore_axis_name="s"),
        scratch_types=[
            pltpu.VMEM((TOPK, CH), jnp.int32),
            pltpu.VMEM((TOPK, CH), jnp.int32),
            pltpu.VMEM((TOPK, CH), jnp.float32),
            pltpu.VMEM((TOPK * CH + 16,), jnp.int32),
            pltpu.VMEM((TOPK * CH + 16,), jnp.float32),
            pltpu.VMEM((CH * TOPK, 2 * DH), jnp.float32),
            pltpu.VMEM((CH, DH), jnp.float32),
            pltpu.SemaphoreType.DMA,
        ],
    )(_sc_body)


def _o_body(ctx_ref, wo_ref, out_ref):
    acc = jnp.dot(ctx_ref[0], wo_ref[0], preferred_element_type=jnp.float32)
    for h in range(1, QH):
        acc += jnp.dot(ctx_ref[h], wo_ref[h],
                       preferred_element_type=jnp.float32)
    out_ref[...] = acc


def _stage_d(ctx3, Wo):
    return pl.pallas_call(
        _o_body,
        grid=(NBLK,),
        in_specs=[
            pl.BlockSpec((QH, SB, DH), lambda i: (0, i, 0)),
            pl.BlockSpec((QH, DH, DM), lambda i: (0, 0, 0)),
        ],
        out_specs=pl.BlockSpec((SB, DM), lambda i: (i, 0)),
        out_shape=jax.ShapeDtypeStruct((S, DM), jnp.float32),
    )(ctx3, Wo.reshape(QH, DH, DM))


def kernel(x, Wq, Wk, Wv, Wo):
    x2 = x.reshape(S, DM)
    cos = jnp.asarray(_COS_NP)
    sin = jnp.asarray(_SIN_NP)
    v_tab, k1, k2 = _stage_a(x2, Wk, Wv, cos, sin)
    w, fidx = _stage_b(x2, Wq, k1, k2, cos, sin)
    v_pack = v_tab.reshape(KVH * S // 2, 2 * DH)
    ctx = _make_sc_gather()(fidx, w, v_pack)
    ctx3 = ctx.reshape(QH, S, DH)
    out = _stage_d(ctx3, Wo)
    return out.reshape(B, S, DM)


# speedup vs baseline: 32.3418x; 1.0033x over previous
import functools
import math

import numpy as np
import jax
import jax.numpy as jnp
from jax import lax
from jax.experimental import pallas as pl
from jax.experimental.pallas import tpu as pltpu
from jax.experimental.pallas import tpu_sc as plsc

B, S, DM = 1, 4096, 768
QH, KVH = 12, 4
DH = DM // QH
HALF = DH // 2
M = 64
TOPK = 4

SB = 512
NBLK = S // SB

_inv_freq = 1.0 / (10000.0 ** (np.arange(0, HALF, dtype=np.float32) / HALF))
_freqs = np.outer(np.arange(S, dtype=np.float32), _inv_freq)
_COS_NP = np.cos(_freqs).astype(np.float32)
_SIN_NP = np.sin(_freqs).astype(np.float32)


def _kv_body(x_ref, wk_ref, wv_ref, cos_ref, sin_ref, v_ref, k1_ref, k2_ref):
    i = pl.program_id(0)
    xb = x_ref[...]
    cos = cos_ref[...]
    sin = sin_ref[...]

    for h in range(KVH):
        kh = jnp.dot(xb, wk_ref[:, h * DH:(h + 1) * DH],
                     preferred_element_type=jnp.float32)
        kh1 = kh[:, :HALF] * cos - kh[:, HALF:] * sin
        kh2 = kh[:, HALF:] * cos + kh[:, :HALF] * sin
        c1 = jnp.concatenate(
            [jnp.sum(kh1[m * M:(m + 1) * M], axis=0, keepdims=True)
             for m in range(SB // M)], axis=0)
        c2 = kh2[0 * M:1 * M]
        for m in range(1, SB // M):
            c2 = c2 + kh2[m * M:(m + 1) * M]
        k1_ref[h] = c1

        @pl.when(i == 0)
        def _(h=h, c2=c2):
            k2_ref[h] = c2

        @pl.when(i != 0)
        def _(h=h, c2=c2):
            k2_ref[h] += c2

        v_ref[h] = jnp.dot(xb, wv_ref[:, h * DH:(h + 1) * DH],
                           preferred_element_type=jnp.float32)


def _stage_a(x2, Wk, Wv, cos, sin):
    return pl.pallas_call(
        _kv_body,
        grid=(NBLK,),
        in_specs=[
            pl.BlockSpec((SB, DM), lambda i: (i, 0)),
            pl.BlockSpec((DM, KVH * DH), lambda i: (0, 0)),
            pl.BlockSpec((DM, KVH * DH), lambda i: (0, 0)),
            pl.BlockSpec((SB, HALF), lambda i: (i, 0)),
            pl.BlockSpec((SB, HALF), lambda i: (i, 0)),
        ],
        out_specs=[
            pl.BlockSpec((KVH, SB, DH), lambda i: (0, i, 0)),
            pl.BlockSpec((KVH, SB // M, HALF), lambda i: (0, i, 0)),
            pl.BlockSpec((KVH, M, HALF), lambda i: (0, 0, 0)),
        ],
        out_shape=[
            jax.ShapeDtypeStruct((KVH, S, DH), jnp.float32),
            jax.ShapeDtypeStruct((KVH, M, HALF), jnp.float32),
            jax.ShapeDtypeStruct((KVH, M, HALF), jnp.float32),
        ],
    )(x2, Wk, Wv, cos, sin)


def _top4_t(s, se, n):
    io = lax.broadcasted_iota(jnp.int32, s.shape, 0)
    vals, idxs = [], []
    for _ in range(TOPK):
        m = jnp.max(s, axis=0, keepdims=True)
        idx = jnp.min(jnp.where(s == m, io, n), axis=0, keepdims=True)
        hit = io == idx
        if se is None:
            vals.append(m)
        else:
            vals.append(jnp.sum(jnp.where(hit, se, 0.0), axis=0,
                                keepdims=True))
        idxs.append(idx)
        s = jnp.where(hit, -jnp.inf, s)
    return jnp.concatenate(vals, axis=0), jnp.concatenate(idxs, axis=0)


def _sel4_t(tab, sel):
    out = jnp.zeros_like(tab)
    for t in range(TOPK):
        out = jnp.where(sel == t, tab[t:t + 1], out)
    return out


def _q_body(x_ref, wq_ref, k1_ref, k2_ref, cos_ref, sin_ref, w_ref, fidx_ref):
    xb = x_ref[...]
    xq = jnp.dot(xb, wq_ref[...], preferred_element_type=jnp.float32)
    cos = cos_ref[...]
    sin = sin_ref[...]
    dn = (((1,), (1,)), ((), ()))
    for h in range(QH):
        qh = xq[:, h * DH:(h + 1) * DH]
        q1 = qh[:, :HALF] * cos - qh[:, HALF:] * sin
        q2 = qh[:, HALF:] * cos + qh[:, :HALF] * sin
        kv = h // (QH // KVH)
        k1h = k1_ref[kv]
        k2h = k2_ref[kv]
        s1 = lax.dot_general(k1h, q1, dn, preferred_element_type=jnp.float32)
        s2 = lax.dot_general(k2h, q2, dn, preferred_element_type=jnp.float32)
        s1e = lax.dot_general(k1h, q1, dn, precision=lax.Precision.HIGHEST,
                              preferred_element_type=jnp.float32)
        s2e = lax.dot_general(k2h, q2, dn, precision=lax.Precision.HIGHEST,
                              preferred_element_type=jnp.float32)
        v1, i1 = _top4_t(s1, s1e, M)
        v2, i2 = _top4_t(s2, s2e, M)
        comb = jnp.concatenate(
            [v1[t:t + 1] + v2 for t in range(TOPK)], axis=0)
        cv, sel = _top4_t(comb, None, TOPK * TOPK)
        a = cv * (1.0 / math.sqrt(DH))
        e = jnp.exp(a - jnp.max(a, axis=0, keepdims=True))
        w = e / jnp.sum(e, axis=0, keepdims=True)
        idx1 = sel // TOPK
        idx2 = sel % TOPK
        row = _sel4_t(i1, idx1)
        col = _sel4_t(i2, idx2)
        w_ref[h] = w
        fidx_ref[h] = row * M + col + kv * S


def _stage_b(x2, Wq, k1, k2, cos, sin):
    return pl.pallas_call(
        _q_body,
        grid=(NBLK,),
        in_specs=[
            pl.BlockSpec((SB, DM), lambda i: (i, 0)),
            pl.BlockSpec((DM, QH * DH), lambda i: (0, 0)),
            pl.BlockSpec((KVH, M, HALF), lambda i: (0, 0, 0)),
            pl.BlockSpec((KVH, M, HALF), lambda i: (0, 0, 0)),
            pl.BlockSpec((SB, HALF), lambda i: (i, 0)),
            pl.BlockSpec((SB, HALF), lambda i: (i, 0)),
        ],
        out_specs=[
            pl.BlockSpec((QH, TOPK, SB), lambda i: (0, 0, i)),
            pl.BlockSpec((QH, TOPK, SB), lambda i: (0, 0, i)),
        ],
        out_shape=[
            jax.ShapeDtypeStruct((QH, TOPK, S), jnp.float32),
            jax.ShapeDtypeStruct((QH, TOPK, S), jnp.int32),
        ],
    )(x2, Wq, k1, k2, cos, sin)


NQ_TOT = QH * S
NW = 32
NQ_W = NQ_TOT // NW
CH = 128
NCHUNK = NQ_W // CH
GSUB = (CH * TOPK) // 128


HCH = CH // 2


def _sc_body(fidx_hbm, w_hbm, vtab_hbm, out_hbm,
             idx_raw, idx2_v, w2d_v, off_v, w_v, rows0, rows1, out_v,
             sem0, sem1):
    wid = lax.axis_index("s") * 2 + lax.axis_index("c")
    qbase = wid * NQ_W
    rows = (rows0, rows1)
    sems = (sem0, sem1)

    def block(bi, carry):
        q0 = qbase + bi * CH
        h = q0 // S
        s0 = pl.multiple_of(q0 % S, CH)
        pltpu.sync_copy(fidx_hbm.at[h, :, pl.ds(s0, CH)], idx_raw)
        pltpu.sync_copy(w_hbm.at[h, :, pl.ds(s0, CH)], w2d_v)
        for k in range(TOPK):
            for j in range(CH // 16):
                sl = pl.ds(j * 16, 16)
                raw = idx_raw[k, sl]
                idx2_v[j // (HCH // 16), k,
                       pl.ds((j % (HCH // 16)) * 16, 16)] = raw >> 1
                off_v[pl.ds(k * CH + j * 16, 16)] = (raw & 1) * DH
                w_v[pl.ds(k * CH + j * 16, 16)] = w2d_v[k, sl]
        cps = [[pltpu.async_copy(vtab_hbm.at[idx2_v.at[half, k]],
                                 rows[half].at[pl.ds(k * HCH, HCH)],
                                 sems[half])
                for k in range(TOPK)] for half in range(2)]
        for half in range(2):
            for c in cps[half]:
                c.wait()
            rbuf = rows[half]

            def qloop(l, c2, half=half, rbuf=rbuf):
                qi = half * HCH + l
                acc = [None] * (DH // 16)
                for kk in range(TOPK):
                    wk = w_v[pl.ds(kk * CH + qi, 16)][0]
                    ofk = off_v[pl.ds(kk * CH + qi, 16)][0]
                    r = kk * HCH + l
                    for dv in range(DH // 16):
                        term = wk * rbuf[r, pl.ds(ofk + dv * 16, 16)]
                        acc[dv] = term if kk == 0 else acc[dv] + term
                for dv in range(DH // 16):
                    out_v[l, pl.ds(dv * 16, 16)] = acc[dv]
                return c2

            lax.fori_loop(0, HCH, qloop, 0)
            pltpu.sync_copy(out_v,
                            out_hbm.at[pl.ds(q0 + half * HCH, HCH)])
        return carry

    lax.fori_loop(0, NCHUNK, block, 0)


@functools.lru_cache(maxsize=1)
def _make_sc_gather():
    return functools.partial(
        pl.kernel,
        out_type=jax.ShapeDtypeStruct((NQ_TOT, DH), jnp.float32),
        mesh=plsc.VectorSubcoreMesh(core_axis_name="c", subcore_axis_name="s"),
        scratch_types=[
            pltpu.VMEM((TOPK, CH), jnp.int32),
            pltpu.VMEM((2, TOPK, HCH), jnp.int32),
            pltpu.VMEM((TOPK, CH), jnp.float32),
            pltpu.VMEM((TOPK * CH + 16,), jnp.int32),
            pltpu.VMEM((TOPK * CH + 16,), jnp.float32),
            pltpu.VMEM((HCH * TOPK, 2 * DH), jnp.float32),
            pltpu.VMEM((HCH * TOPK, 2 * DH), jnp.float32),
            pltpu.VMEM((HCH, DH), jnp.float32),
            pltpu.SemaphoreType.DMA,
            pltpu.SemaphoreType.DMA,
        ],
    )(_sc_body)


def _o_body(ctx_ref, wo_ref, out_ref):
    acc = jnp.dot(ctx_ref[0], wo_ref[0], preferred_element_type=jnp.float32)
    for h in range(1, QH):
        acc += jnp.dot(ctx_ref[h], wo_ref[h],
                       preferred_element_type=jnp.float32)
    out_ref[...] = acc


def _stage_d(ctx3, Wo):
    return pl.pallas_call(
        _o_body,
        grid=(NBLK,),
        in_specs=[
            pl.BlockSpec((QH, SB, DH), lambda i: (0, i, 0)),
            pl.BlockSpec((QH, DH, DM), lambda i: (0, 0, 0)),
        ],
        out_specs=pl.BlockSpec((SB, DM), lambda i: (i, 0)),
        out_shape=jax.ShapeDtypeStruct((S, DM), jnp.float32),
    )(ctx3, Wo.reshape(QH, DH, DM))


def kernel(x, Wq, Wk, Wv, Wo):
    x2 = x.reshape(S, DM)
    cos = jnp.asarray(_COS_NP)
    sin = jnp.asarray(_SIN_NP)
    v_tab, k1, k2 = _stage_a(x2, Wk, Wv, cos, sin)
    w, fidx = _stage_b(x2, Wq, k1, k2, cos, sin)
    v_pack = v_tab.reshape(KVH * S // 2, 2 * DH)
    ctx = _make_sc_gather()(fidx, w, v_pack)
    ctx3 = ctx.reshape(QH, S, DH)
    out = _stage_d(ctx3, Wo)
    return out.reshape(B, S, DM)
